# baseline (speedup 1.0000x reference)
import jax
import jax.numpy as jnp
from jax import lax
from jax.experimental import pallas as pl
from jax.experimental.pallas import tpu as pltpu, tpu_sc as plsc

_NC = 2
_NS = 16
_NW = _NC * _NS
_CR = 16


def _make_sc_kernel(B, T, D, dtype):
    assert B == 4, "pipeline depth is wired for B == 4"
    seq_per_w = T // _NW
    n_tc = seq_per_w // _CR
    n_lane = D // 16
    mesh = plsc.VectorSubcoreMesh(core_axis_name="c", subcore_axis_name="s")

    def body(x_hbm, tab_hbm, out_hbm,
             tb0, tb1, xb0, xb1, xb2, xb3,
             st0, st1, si0, si1, si2, si3, so0, so1, so2, so3):
        wid = lax.axis_index("s") * _NC + lax.axis_index("c")
        seq0 = wid * seq_per_w
        tbufs = (tb0, tb1)
        stab = (st0, st1)
        xbufs = (xb0, xb1, xb2, xb3)
        sin = (si0, si1, si2, si3)
        sout = (so0, so1, so2, so3)

        def row0(c, b):
            return b * T + seq0 + c * _CR

        def load_tab(c, tp):
            pltpu.async_copy(
                tab_hbm.at[pl.ds(seq0 + c * _CR, _CR)], tbufs[tp], stab[tp])

        def wait_tab(c, tp):
            pltpu.make_async_copy(
                tab_hbm.at[pl.ds(seq0 + c * _CR, _CR)], tbufs[tp],
                stab[tp]).wait()

        def load(c, b, p):
            pltpu.async_copy(
                x_hbm.at[pl.ds(row0(c, b), _CR)], xbufs[p], sin[p])

        def wait_load(c, b, p):
            pltpu.make_async_copy(
                x_hbm.at[pl.ds(row0(c, b), _CR)], xbufs[p], sin[p]).wait()

        def store(c, b, p):
            pltpu.async_copy(
                xbufs[p], out_hbm.at[pl.ds(row0(c, b), _CR)], sout[p])

        def wait_store(c, b, p):
            pltpu.make_async_copy(
                xbufs[p], out_hbm.at[pl.ds(row0(c, b), _CR)], sout[p]).wait()

        load_tab(0, 0)
        load(0, 0, 0)
        load(0, 1, 1)

        @pl.loop(0, n_tc // 2)
        def _pair(cc):
            for half in range(2):
                c = cc * 2 + half
                @pl.when(c < n_tc - 1)
                def _():
                    load_tab(c + 1, 1 - half)
                wait_tab(c, half)
                for b in range(B):
                    if b < B - 2:
                        if half == 0 and b == 0:
                            @pl.when(c > 0)
                            def _():
                                wait_store(c - 1, b + 2, b + 2)
                                load(c, b + 2, b + 2)
                            @pl.when(c == 0)
                            def _():
                                load(c, b + 2, b + 2)
                        elif half == 0 and b == 1:
                            @pl.when(c > 0)
                            def _():
                                wait_store(c - 1, b + 2, b + 2)
                            load(c, b + 2, b + 2)
                        else:
                            wait_store(c - 1, b + 2, b + 2)
                            load(c, b + 2, b + 2)
                    else:
                        @pl.when(c < n_tc - 1)
                        def _():
                            wait_store(c, b - 2, b - 2)
                            load(c + 1, b - 2, b - 2)
                    wait_load(c, b, b)

                    @plsc.parallel_loop(0, _CR, unroll=2)
                    def _add(r):
                        for i in range(n_lane):
                            sl = pl.ds(i * 16, 16)
                            v = tbufs[half][r, sl]
                            plsc.addupdate(xbufs[b].at[r, sl], v)

                    store(c, b, b)

        wait_store(n_tc - 1, 0, 0)
        wait_store(n_tc - 1, 1, 1)
        wait_store(n_tc - 1, 2, 2)
        wait_store(n_tc - 1, 3, 3)

    return pl.kernel(
        body,
        out_type=jax.ShapeDtypeStruct((B * T, D), dtype),
        mesh=mesh,
        scratch_types=[
            pltpu.VMEM((_CR, D), dtype),
            pltpu.VMEM((_CR, D), dtype),
            pltpu.VMEM((_CR, D), dtype),
            pltpu.VMEM((_CR, D), dtype),
            pltpu.VMEM((_CR, D), dtype),
            pltpu.VMEM((_CR, D), dtype),
        ] + [pltpu.SemaphoreType.DMA] * 10,
    )


def kernel(x, pos_table):
    B, T, D = x.shape
    x2 = x.reshape(B * T, D)
    out = _make_sc_kernel(B, T, D, x.dtype)(x2, pos_table[:T])
    return out.reshape(B, T, D)


# speedup vs baseline: 1.8682x; 1.8682x over previous
import jax
import jax.numpy as jnp
from jax.experimental import pallas as pl

_BS = 1024


def _add_body(x_ref, p_ref, o_ref):
    o_ref[...] = x_ref[...] + p_ref[...]


def kernel(x, pos_table):
    B, T, D = x.shape
    pe = pos_table[:T]
    n_seq = T // _BS
    return pl.pallas_call(
        _add_body,
        grid=(n_seq, B),
        in_specs=[
            pl.BlockSpec((1, _BS, D), lambda s, b: (b, s, 0)),
            pl.BlockSpec((_BS, D), lambda s, b: (s, 0)),
        ],
        out_specs=pl.BlockSpec((1, _BS, D), lambda s, b: (b, s, 0)),
        out_shape=jax.ShapeDtypeStruct((B, T, D), x.dtype),
    )(x, pe)


# speedup vs baseline: 1.9458x; 1.0415x over previous
import jax
import jax.numpy as jnp
from jax.experimental import pallas as pl

_BS = 2048


def _add_body(x_ref, p_ref, o_ref):
    o_ref[...] = x_ref[...] + p_ref[...]


def kernel(x, pos_table):
    B, T, D = x.shape
    pe = pos_table[:T]
    n_seq = T // _BS
    return pl.pallas_call(
        _add_body,
        grid=(n_seq, B),
        in_specs=[
            pl.BlockSpec((1, _BS, D), lambda s, b: (b, s, 0)),
            pl.BlockSpec((_BS, D), lambda s, b: (s, 0)),
        ],
        out_specs=pl.BlockSpec((1, _BS, D), lambda s, b: (b, s, 0)),
        out_shape=jax.ShapeDtypeStruct((B, T, D), x.dtype),
    )(x, pe)
